# baseline (speedup 1.0000x reference)
import jax
import jax.numpy as jnp
from jax.experimental import pallas as pl
from jax.experimental.pallas import tpu as pltpu

F32 = jnp.float32

_SHIFTS4 = tuple((dy, dx) for dy in range(2) for dx in range(2))
_SHIFTS9 = tuple((ky, kx) for ky in range(3) for kx in range(3))


def _s2d_pad(x, w_to):
    N, H, W, C = x.shape
    xp = jnp.pad(x, ((0, 0), (1, 1), (1, 1), (0, 0)))
    a, b = (H + 2) // 2, (W + 2) // 2
    xp = xp.reshape(N, a, 2, b, 2, C).transpose(0, 1, 3, 2, 4, 5).reshape(N, a, b, 4 * C)
    return jnp.pad(xp, ((0, 0), (0, 0), (0, w_to - b), (0, 0)))


def _pad1(x, w_to):
    N, H, W, C = x.shape
    return jnp.pad(x, ((0, 0), (1, 1), (1, w_to - W - 1), (0, 0)))


def _prep_s2(w):
    Cout, Cin = w.shape[0], w.shape[1]
    t = jnp.transpose(w, (2, 3, 1, 0))
    t = t.reshape(2, 2, 2, 2, Cin, Cout)
    t = jnp.transpose(t, (0, 2, 1, 3, 4, 5))
    return t.reshape(4, 4 * Cin, Cout)


def _prep_s1(w):
    return jnp.transpose(w, (2, 3, 1, 0)).reshape(9, w.shape[1], w.shape[0])


def _prep_tconv(wt):
    mats = []
    for py in range(2):
        for px in range(2):
            for t0 in range(2):
                for t1 in range(2):
                    mats.append(wt[:, :, 3 - py - 2 * t0, 3 - px - 2 * t1])
    return jnp.stack(mats)


def _prep_tconv_big(wt):
    Cin, Cout = wt.shape[0], wt.shape[1]
    taps = []
    for ey in range(3):
        for ex in range(3):
            cols = []
            for py in range(2):
                for px in range(2):
                    t0, t1 = ey - py, ex - px
                    if t0 in (0, 1) and t1 in (0, 1):
                        cols.append(wt[:, :, 3 - py - 2 * t0, 3 - px - 2 * t1])
                    else:
                        cols.append(jnp.zeros((Cin, Cout), F32))
            taps.append(jnp.concatenate(cols, axis=1))
    return jnp.stack(taps)


def _conv_taps(xp, w_taps, bias, shifts, Ho, Wo, relu):
    N, Hp, Wp, K = xp.shape
    T, _, Cout = w_taps.shape

    def body(x_ref, w_ref, b_ref, o_ref):
        acc = jnp.zeros((Ho * Wo, Cout), F32)
        for t, (dy, dx) in enumerate(shifts):
            xs = x_ref[0, dy:dy + Ho, dx:dx + Wo, :].reshape(Ho * Wo, K)
            acc = acc + jnp.dot(xs, w_ref[t], preferred_element_type=F32)
        acc = acc + b_ref[:]
        if relu:
            acc = jnp.maximum(acc, 0.0)
        o_ref[0] = acc.reshape(Ho, Wo, Cout)

    return pl.pallas_call(
        body,
        grid=(N,),
        in_specs=[
            pl.BlockSpec((1, Hp, Wp, K), lambda n: (n, 0, 0, 0)),
            pl.BlockSpec((T, K, Cout), lambda n: (0, 0, 0)),
            pl.BlockSpec((1, Cout), lambda n: (0, 0)),
        ],
        out_specs=pl.BlockSpec((1, Ho, Wo, Cout), lambda n: (n, 0, 0, 0)),
        out_shape=jax.ShapeDtypeStruct((N, Ho, Wo, Cout), F32),
    )(xp, w_taps, bias)


def _tconv_phases(xp, w_taps, bias, Ho, Wo, relu):
    N, Hp, Wp, Cin = xp.shape
    Cout = w_taps.shape[2]

    def body(x_ref, w_ref, b_ref, o_ref):
        for p, (py, px) in enumerate(_SHIFTS4):
            acc = jnp.zeros((Ho * Wo, Cout), F32)
            for t0 in range(2):
                for t1 in range(2):
                    ey, ex = py + t0, px + t1
                    xs = x_ref[0, ey:ey + Ho, ex:ex + Wo, :].reshape(Ho * Wo, Cin)
                    acc = acc + jnp.dot(xs, w_ref[p * 4 + t0 * 2 + t1],
                                        preferred_element_type=F32)
            acc = acc + b_ref[:]
            if relu:
                acc = jnp.maximum(acc, 0.0)
            o_ref[0, :, :, p * Cout:(p + 1) * Cout] = acc.reshape(Ho, Wo, Cout)

    return pl.pallas_call(
        body,
        grid=(N,),
        in_specs=[
            pl.BlockSpec((1, Hp, Wp, Cin), lambda n: (n, 0, 0, 0)),
            pl.BlockSpec((16, Cin, Cout), lambda n: (0, 0, 0)),
            pl.BlockSpec((1, Cout), lambda n: (0, 0)),
        ],
        out_specs=pl.BlockSpec((1, Ho, Wo, 4 * Cout), lambda n: (n, 0, 0, 0)),
        out_shape=jax.ShapeDtypeStruct((N, Ho, Wo, 4 * Cout), F32),
    )(xp, w_taps, bias)


def _vq(z, cb, cb_t):
    N = z.shape[0]
    HW, D, Kc = 56 * 56, 32, cb.shape[0]

    def body(z_ref, cb_ref, cbt_ref, q_ref, loss_ref):
        n = pl.program_id(0)
        flat = z_ref[0].reshape(HW, D)
        cb_sq = jnp.sum(cbt_ref[:] ** 2, axis=0, keepdims=True)
        scores = cb_sq - 2.0 * jnp.dot(flat, cbt_ref[:], preferred_element_type=F32)
        iota = jax.lax.broadcasted_iota(jnp.int32, (HW, Kc), 1)
        m = jnp.min(scores, axis=1, keepdims=True)
        idx = jnp.min(jnp.where(scores == m, iota, Kc), axis=1, keepdims=True)
        onehot = (iota == idx).astype(F32)
        q = jax.lax.dot_general(onehot, cb_ref[:], (((1,), (0,)), ((), ())),
                                precision=jax.lax.Precision.HIGHEST,
                                preferred_element_type=F32)
        @pl.when(n == 0)
        def _():
            loss_ref[:] = jnp.zeros((1, 1), F32)
        loss_ref[:] = loss_ref[:] + jnp.sum((q - flat) ** 2).reshape(1, 1)
        q_ref[0] = q.reshape(56, 56, D)

    return pl.pallas_call(
        body,
        grid=(N,),
        in_specs=[
            pl.BlockSpec((1, 56, 56, D), lambda n: (n, 0, 0, 0)),
            pl.BlockSpec((Kc, D), lambda n: (0, 0)),
            pl.BlockSpec((D, Kc), lambda n: (0, 0)),
        ],
        out_specs=[
            pl.BlockSpec((1, 56, 56, D), lambda n: (n, 0, 0, 0)),
            pl.BlockSpec((1, 1), lambda n: (0, 0)),
        ],
        out_shape=[
            jax.ShapeDtypeStruct((N, 56, 56, D), F32),
            jax.ShapeDtypeStruct((1, 1), F32),
        ],
    )(z, cb, cb_t)


def kernel(x, enc_w1, enc_b1, enc_w2, enc_b2, enc_w3, enc_b3, codebook,
           dec_w1, dec_b1, dec_wt1, dec_bt1, dec_wt2, dec_bt2):
    N = x.shape[0]

    w1 = _prep_s2(enc_w1)
    w2 = _prep_s2(enc_w2)
    w3 = _prep_s1(enc_w3)
    w5 = _prep_s1(dec_w1)
    w6 = _prep_tconv(dec_wt1)
    w7 = _prep_tconv_big(dec_wt2)
    b1 = enc_b1.reshape(1, -1)
    b2 = enc_b2.reshape(1, -1)
    b3 = enc_b3.reshape(1, -1)
    b5 = dec_b1.reshape(1, -1)
    b6 = dec_bt1.reshape(1, -1)
    b7 = jnp.tile(dec_bt2, 4).reshape(1, -1)

    xh = jnp.transpose(x, (0, 2, 3, 1))
    h1 = _conv_taps(_s2d_pad(xh, 120), w1, b1, _SHIFTS4, 112, 112, True)
    h2 = _conv_taps(_s2d_pad(h1, 64), w2, b2, _SHIFTS4, 56, 56, True)
    z = _conv_taps(_pad1(h2, 64), w3, b3, _SHIFTS9, 56, 56, False)

    q, loss_sum = _vq(z, codebook, jnp.transpose(codebook))

    h5 = _conv_taps(_pad1(q, 64), w5, b5, _SHIFTS9, 56, 56, True)
    h6p = _tconv_phases(_pad1(h5, 64), w6, b6, 56, 56, True)
    h6 = h6p.reshape(N, 56, 56, 2, 2, 128).transpose(0, 1, 3, 2, 4, 5)
    h6 = h6.reshape(N, 112, 112, 128)
    o7 = _conv_taps(_pad1(h6, 120), w7, b7, _SHIFTS9, 112, 112, False)
    out = o7.reshape(N, 112, 112, 2, 2, 3).transpose(0, 1, 3, 2, 4, 5)
    out = out.reshape(N, 224, 224, 3).transpose(0, 3, 1, 2)

    loss = (1.25 / (N * 56 * 56 * 32)) * loss_sum[0, 0]
    return (out, loss)


# speedup vs baseline: 1.4575x; 1.4575x over previous
import jax
import jax.numpy as jnp
from jax.experimental import pallas as pl
from jax.experimental.pallas import tpu as pltpu

F32 = jnp.float32
BF16 = jnp.bfloat16


def _prep_conv1(w):
    wt = jnp.transpose(w, (2, 3, 1, 0))
    mats = []
    for v in range(2):
        for u in range(2):
            for dy in range(2):
                for dx in range(2):
                    m = jnp.zeros((4, 4, 3, 128), F32)
                    for p4y in range(4):
                        ky = 4 * dy + p4y - 2 * v
                        if not 0 <= ky < 4:
                            continue
                        for p4x in range(4):
                            kx = 4 * dx + p4x - 2 * u
                            if 0 <= kx < 4:
                                m = m.at[p4y, p4x].set(wt[ky, kx])
                    mats.append(m.reshape(48, 128))
    return jnp.stack(mats)


def _prep_conv2(w):
    wt = jnp.transpose(w, (2, 3, 1, 0))
    mats = []
    for py in range(2):
        for px in range(2):
            for dy in range(2):
                for dx in range(2):
                    mats.append(wt[2 * dy + py, 2 * dx + px])
    return jnp.stack(mats)


def _prep_s1(w):
    return jnp.transpose(w, (2, 3, 1, 0)).reshape(9, w.shape[1], w.shape[0])


def _prep_tconv(wt):
    mats = []
    for py in range(2):
        for px in range(2):
            for t0 in range(2):
                for t1 in range(2):
                    mats.append(wt[:, :, 3 - py - 2 * t0, 3 - px - 2 * t1])
    return jnp.stack(mats)


def _prep_tconv_big(wt):
    Cin, Cout = wt.shape[0], wt.shape[1]
    taps = []
    for ey in range(3):
        for ex in range(3):
            cols = []
            for qy in range(2):
                for qx in range(2):
                    t0, t1 = ey - qy, ex - qx
                    if t0 in (0, 1) and t1 in (0, 1):
                        cols.append(wt[:, :, 3 - qy - 2 * t0, 3 - qx - 2 * t1])
                    else:
                        cols.append(jnp.zeros((Cin, Cout), F32))
            taps.append(jnp.concatenate(cols, axis=1))
    return jnp.stack(taps)


_T2 = {0: (1, 0), 1: (0, 1), 2: (1, 1), 3: (0, 2)}


def _body(xs_ref, w1_ref, b1_ref, w2_ref, b2_ref, w3_ref, b3_ref,
          cb_ref, cbt_ref, w5_ref, b5_ref, w6_ref, b6_ref, w7_ref, b7_ref,
          o_ref, loss_ref, p1_ref, h2_ref, qp_ref, h5_ref, p6_ref):
    n = pl.program_id(0)

    @pl.when(n == 0)
    def _():
        p1_ref[...] = jnp.zeros((2, 2, 58, 64, 128), F32)
        h2_ref[...] = jnp.zeros((58, 64, 128), F32)
        qp_ref[...] = jnp.zeros((58, 64, 32), F32)
        h5_ref[...] = jnp.zeros((58, 64, 128), BF16)
        p6_ref[...] = jnp.zeros((2, 2, 58, 64, 128), BF16)
        loss_ref[...] = jnp.zeros((1, 1), F32)

    for v in range(2):
        for u in range(2):
            acc = jnp.zeros((3136, 128), F32)
            for dy in range(2):
                for dx in range(2):
                    xs = xs_ref[0, dy:dy + 56, dx:dx + 56, :].reshape(3136, 48)
                    acc = acc + jnp.dot(xs, w1_ref[((v * 2 + u) * 2 + dy) * 2 + dx],
                                        preferred_element_type=F32)
            acc = jnp.maximum(acc + b1_ref[:], 0.0)
            p1_ref[v, u, 1:57, 1:57, :] = acc.reshape(56, 56, 128)

    acc = jnp.zeros((3136, 128), F32)
    for py in range(2):
        for px in range(2):
            for dy in range(2):
                for dx in range(2):
                    xs = p1_ref[py ^ 1, px ^ 1, dy + py:dy + py + 56,
                                dx + px:dx + px + 56, :].reshape(3136, 128)
                    acc = acc + jnp.dot(xs, w2_ref[((py * 2 + px) * 2 + dy) * 2 + dx],
                                        preferred_element_type=F32)
    acc = jnp.maximum(acc + b2_ref[:], 0.0)
    h2_ref[1:57, 1:57, :] = acc.reshape(56, 56, 128)

    acc = jnp.zeros((3136, 32), F32)
    for ky in range(3):
        for kx in range(3):
            xs = h2_ref[ky:ky + 56, kx:kx + 56, :].reshape(3136, 128)
            acc = acc + jnp.dot(xs, w3_ref[ky * 3 + kx], preferred_element_type=F32)
    z = acc + b3_ref[:]

    cb_sq = jnp.sum(cbt_ref[:] ** 2, axis=0, keepdims=True)
    loss_part = jnp.zeros((1, 1), F32)
    for c in range(4):
        zc = z[c * 784:(c + 1) * 784, :]
        scores = cb_sq - 2.0 * jnp.dot(zc, cbt_ref[:], preferred_element_type=F32)
        iota = jax.lax.broadcasted_iota(jnp.int32, (784, 1024), 1)
        m = jnp.min(scores, axis=1, keepdims=True)
        idx = jnp.min(jnp.where(scores == m, iota, 1024), axis=1, keepdims=True)
        onehot = (iota == idx).astype(F32)
        qc = jax.lax.dot_general(onehot, cb_ref[:], (((1,), (0,)), ((), ())),
                                 precision=jax.lax.Precision.HIGHEST,
                                 preferred_element_type=F32)
        loss_part = loss_part + jnp.sum((qc - zc) ** 2).reshape(1, 1)
        qp_ref[1 + c * 14:1 + (c + 1) * 14, 1:57, :] = qc.reshape(14, 56, 32)
    loss_ref[:] = loss_ref[:] + loss_part

    acc = jnp.zeros((3136, 128), F32)
    for ky in range(3):
        for kx in range(3):
            xs = qp_ref[ky:ky + 56, kx:kx + 56, :].reshape(3136, 32)
            acc = acc + jnp.dot(xs.astype(BF16), w5_ref[ky * 3 + kx],
                                preferred_element_type=F32)
    acc = jnp.maximum(acc + b5_ref[:], 0.0)
    h5_ref[1:57, 1:57, :] = acc.reshape(56, 56, 128).astype(BF16)

    for py in range(2):
        for px in range(2):
            acc = jnp.zeros((3136, 128), F32)
            for t0 in range(2):
                for t1 in range(2):
                    xs = h5_ref[py + t0:py + t0 + 56,
                                px + t1:px + t1 + 56, :].reshape(3136, 128)
                    acc = acc + jnp.dot(xs, w6_ref[((py * 2 + px) * 2 + t0) * 2 + t1],
                                        preferred_element_type=F32)
            acc = jnp.maximum(acc + b6_ref[:], 0.0)
            p6_ref[py, px, 1:57, 1:57, :] = acc.reshape(56, 56, 128).astype(BF16)

    for t in range(2):
        for u in range(2):
            acc = jnp.zeros((3136, 12), F32)
            for ey in range(3):
                wy, jy = _T2[t + ey]
                for ex in range(3):
                    wx, jx = _T2[u + ex]
                    xs = p6_ref[wy, wx, jy:jy + 56, jx:jx + 56, :].reshape(3136, 128)
                    acc = acc + jnp.dot(xs, w7_ref[ey * 3 + ex],
                                        preferred_element_type=F32)
            acc = acc + b7_ref[:]
            o_ref[0, t, u] = acc.reshape(56, 56, 12).astype(BF16)


def kernel(x, enc_w1, enc_b1, enc_w2, enc_b2, enc_w3, enc_b3, codebook,
           dec_w1, dec_b1, dec_wt1, dec_bt1, dec_wt2, dec_bt2):
    N = x.shape[0]

    w1 = _prep_conv1(enc_w1)
    w2 = _prep_conv2(enc_w2)
    w3 = _prep_s1(enc_w3)
    w5 = _prep_s1(dec_w1).astype(BF16)
    w6 = _prep_tconv(dec_wt1).astype(BF16)
    w7 = _prep_tconv_big(dec_wt2).astype(BF16)
    b1 = enc_b1.reshape(1, -1)
    b2 = enc_b2.reshape(1, -1)
    b3 = enc_b3.reshape(1, -1)
    b5 = dec_b1.reshape(1, -1)
    b6 = dec_bt1.reshape(1, -1)
    b7 = jnp.tile(dec_bt2, 4).reshape(1, -1)

    xh = jnp.transpose(x, (0, 2, 3, 1))
    xp = jnp.pad(xh, ((0, 0), (1, 3), (1, 3), (0, 0)))
    xs4 = xp.reshape(N, 57, 4, 57, 4, 3).transpose(0, 1, 3, 2, 4, 5)
    xs4 = jnp.pad(xs4.reshape(N, 57, 57, 48), ((0, 0), (0, 0), (0, 7), (0, 0)))

    out, loss_sum = pl.pallas_call(
        _body,
        grid=(N,),
        in_specs=[
            pl.BlockSpec((1, 57, 64, 48), lambda n: (n, 0, 0, 0)),
            pl.BlockSpec((16, 48, 128), lambda n: (0, 0, 0)),
            pl.BlockSpec((1, 128), lambda n: (0, 0)),
            pl.BlockSpec((16, 128, 128), lambda n: (0, 0, 0)),
            pl.BlockSpec((1, 128), lambda n: (0, 0)),
            pl.BlockSpec((9, 128, 32), lambda n: (0, 0, 0)),
            pl.BlockSpec((1, 32), lambda n: (0, 0)),
            pl.BlockSpec((1024, 32), lambda n: (0, 0)),
            pl.BlockSpec((32, 1024), lambda n: (0, 0)),
            pl.BlockSpec((9, 32, 128), lambda n: (0, 0, 0)),
            pl.BlockSpec((1, 128), lambda n: (0, 0)),
            pl.BlockSpec((16, 128, 128), lambda n: (0, 0, 0)),
            pl.BlockSpec((1, 128), lambda n: (0, 0)),
            pl.BlockSpec((9, 128, 12), lambda n: (0, 0, 0)),
            pl.BlockSpec((1, 12), lambda n: (0, 0)),
        ],
        out_specs=[
            pl.BlockSpec((1, 2, 2, 56, 56, 12), lambda n: (n, 0, 0, 0, 0, 0)),
            pl.BlockSpec((1, 1), lambda n: (0, 0)),
        ],
        out_shape=[
            jax.ShapeDtypeStruct((N, 2, 2, 56, 56, 12), BF16),
            jax.ShapeDtypeStruct((1, 1), F32),
        ],
        scratch_shapes=[
            pltpu.VMEM((2, 2, 58, 64, 128), F32),
            pltpu.VMEM((58, 64, 128), F32),
            pltpu.VMEM((58, 64, 32), F32),
            pltpu.VMEM((58, 64, 128), BF16),
            pltpu.VMEM((2, 2, 58, 64, 128), BF16),
        ],
    )(xs4, w1, b1, w2, b2, w3, b3, codebook, jnp.transpose(codebook),
      w5, b5, w6, b6, w7, b7)

    o = out.astype(F32).reshape(N, 2, 2, 56, 56, 2, 2, 3)
    o = o.transpose(0, 3, 1, 5, 4, 2, 6, 7).reshape(N, 224, 224, 3)
    o = o.transpose(0, 3, 1, 2)

    loss = (1.25 / (N * 56 * 56 * 32)) * loss_sum[0, 0]
    return (o, loss)


# speedup vs baseline: 1.5357x; 1.0536x over previous
import functools

import jax
import jax.numpy as jnp
from jax import lax
from jax.experimental import pallas as pl
from jax.experimental.pallas import tpu as pltpu
from jax.experimental.pallas import tpu_sc as plsc

F32 = jnp.float32
BF16 = jnp.bfloat16


def _prep_conv1(w):
    wt = jnp.transpose(w, (2, 3, 1, 0))
    mats = []
    for v in range(2):
        for u in range(2):
            for dy in range(2):
                for dx in range(2):
                    m = jnp.zeros((4, 4, 3, 128), F32)
                    for p4y in range(4):
                        ky = 4 * dy + p4y - 2 * v
                        if not 0 <= ky < 4:
                            continue
                        for p4x in range(4):
                            kx = 4 * dx + p4x - 2 * u
                            if 0 <= kx < 4:
                                m = m.at[p4y, p4x].set(wt[ky, kx])
                    mats.append(m.reshape(48, 128))
    return jnp.stack(mats)


def _prep_conv2(w):
    wt = jnp.transpose(w, (2, 3, 1, 0))
    mats = []
    for py in range(2):
        for px in range(2):
            for dy in range(2):
                for dx in range(2):
                    mats.append(wt[2 * dy + py, 2 * dx + px])
    return jnp.stack(mats)


def _prep_s1(w):
    return jnp.transpose(w, (2, 3, 1, 0)).reshape(9, w.shape[1], w.shape[0])


def _prep_tconv(wt):
    mats = []
    for py in range(2):
        for px in range(2):
            for t0 in range(2):
                for t1 in range(2):
                    mats.append(wt[:, :, 3 - py - 2 * t0, 3 - px - 2 * t1])
    return jnp.stack(mats)


def _prep_tconv_big(wt):
    Cin, Cout = wt.shape[0], wt.shape[1]
    taps = []
    for ey in range(3):
        for ex in range(3):
            cols = []
            for qy in range(2):
                for qx in range(2):
                    t0, t1 = ey - qy, ex - qx
                    if t0 in (0, 1) and t1 in (0, 1):
                        cols.append(wt[:, :, 3 - qy - 2 * t0, 3 - qx - 2 * t1])
                    else:
                        cols.append(jnp.zeros((Cin, Cout), F32))
            taps.append(jnp.concatenate(cols, axis=1))
    return jnp.stack(taps)


def _enc_body(xs_ref, w1_ref, b1_ref, w2_ref, b2_ref, w3_ref, b3_ref,
              cbt_ref, idx_ref, loss_ref, p1_ref, h2_ref):
    n = pl.program_id(0)

    @pl.when(n == 0)
    def _():
        p1_ref[...] = jnp.zeros((2, 2, 58, 64, 128), F32)
        h2_ref[...] = jnp.zeros((58, 64, 128), F32)
        loss_ref[...] = jnp.zeros((1, 1), F32)

    for v in range(2):
        for u in range(2):
            acc = jnp.zeros((3136, 128), F32)
            for dy in range(2):
                for dx in range(2):
                    xs = xs_ref[0, dy:dy + 56, dx:dx + 56, :].reshape(3136, 48)
                    acc = acc + jnp.dot(xs, w1_ref[((v * 2 + u) * 2 + dy) * 2 + dx],
                                        preferred_element_type=F32)
            acc = jnp.maximum(acc + b1_ref[:], 0.0)
            p1_ref[v, u, 1:57, 1:57, :] = acc.reshape(56, 56, 128)

    acc = jnp.zeros((3136, 128), F32)
    for py in range(2):
        for px in range(2):
            for dy in range(2):
                for dx in range(2):
                    xs = p1_ref[py ^ 1, px ^ 1, dy + py:dy + py + 56,
                                dx + px:dx + px + 56, :].reshape(3136, 128)
                    acc = acc + jnp.dot(xs, w2_ref[((py * 2 + px) * 2 + dy) * 2 + dx],
                                        preferred_element_type=F32)
    acc = jnp.maximum(acc + b2_ref[:], 0.0)
    h2_ref[1:57, 1:57, :] = acc.reshape(56, 56, 128)

    acc = jnp.zeros((3136, 32), F32)
    for ky in range(3):
        for kx in range(3):
            xs = h2_ref[ky:ky + 56, kx:kx + 56, :].reshape(3136, 128)
            acc = acc + jnp.dot(xs, w3_ref[ky * 3 + kx], preferred_element_type=F32)
    z = acc + b3_ref[:]

    cb_sq = jnp.sum(cbt_ref[:] ** 2, axis=0, keepdims=True)
    loss_part = jnp.sum(z * z).reshape(1, 1)
    for c in range(4):
        zc = z[c * 784:(c + 1) * 784, :]
        scores = cb_sq - 2.0 * jnp.dot(zc, cbt_ref[:], preferred_element_type=F32)
        iota = jax.lax.broadcasted_iota(jnp.int32, (784, 1024), 1)
        m = jnp.min(scores, axis=1, keepdims=True)
        idx = jnp.min(jnp.where(scores == m, iota, 1024), axis=1, keepdims=True)
        loss_part = loss_part + jnp.sum(m).reshape(1, 1)
        idx_ref[0, c * 784:(c + 1) * 784, :] = idx
    loss_ref[:] = loss_ref[:] + loss_part


def _sc_codebook_gather(codebook, idx_flat):
    B = idx_flat.shape[0]
    D = codebook.shape[1]
    info = plsc.get_sparse_core_info()
    NW = info.num_cores * info.num_subcores
    n_chunks = 2
    assert B % (8 * NW * n_chunks) == 0
    b_per_c = B // (NW * n_chunks)
    mesh = plsc.VectorSubcoreMesh(core_axis_name="c", subcore_axis_name="s")

    @functools.partial(
        pl.kernel, mesh=mesh,
        out_type=jax.ShapeDtypeStruct((B, D), F32),
        scratch_types=[
            pltpu.VMEM((b_per_c,), jnp.int32),
            pltpu.VMEM((b_per_c, D), F32),
            pltpu.SemaphoreType.DMA,
        ],
    )
    def k(table_hbm, idx_hbm, out_hbm, idx_v, rows_v, sem):
        wid = lax.axis_index("s") * info.num_cores + lax.axis_index("c")
        for c in range(n_chunks):
            base = (wid * n_chunks + c) * b_per_c
            pltpu.sync_copy(idx_hbm.at[pl.ds(base, b_per_c)], idx_v)
            pltpu.async_copy(table_hbm.at[idx_v], rows_v, sem).wait()
            pltpu.sync_copy(rows_v, out_hbm.at[pl.ds(base, b_per_c)])

    return k(codebook, idx_flat)


_T2 = {0: (1, 0), 1: (0, 1), 2: (1, 1), 3: (0, 2)}


def _dec_body(q_ref, w5_ref, b5_ref, w6_ref, b6_ref, w7_ref, b7_ref,
              o_ref, qp_ref, h5_ref, p6_ref):
    n = pl.program_id(0)

    @pl.when(n == 0)
    def _():
        qp_ref[...] = jnp.zeros((58, 64, 32), F32)
        h5_ref[...] = jnp.zeros((58, 64, 128), BF16)
        p6_ref[...] = jnp.zeros((2, 2, 58, 64, 128), BF16)

    qp_ref[1:57, 1:57, :] = q_ref[0, :, :, 0:32]

    acc = jnp.zeros((3136, 128), F32)
    for ky in range(3):
        for kx in range(3):
            xs = qp_ref[ky:ky + 56, kx:kx + 56, :].reshape(3136, 32)
            acc = acc + jnp.dot(xs.astype(BF16), w5_ref[ky * 3 + kx],
                                preferred_element_type=F32)
    acc = jnp.maximum(acc + b5_ref[:], 0.0)
    h5_ref[1:57, 1:57, :] = acc.reshape(56, 56, 128).astype(BF16)

    for py in range(2):
        for px in range(2):
            acc = jnp.zeros((3136, 128), F32)
            for t0 in range(2):
                for t1 in range(2):
                    xs = h5_ref[py + t0:py + t0 + 56,
                                px + t1:px + t1 + 56, :].reshape(3136, 128)
                    acc = acc + jnp.dot(xs, w6_ref[((py * 2 + px) * 2 + t0) * 2 + t1],
                                        preferred_element_type=F32)
            acc = jnp.maximum(acc + b6_ref[:], 0.0)
            p6_ref[py, px, 1:57, 1:57, :] = acc.reshape(56, 56, 128).astype(BF16)

    for t in range(2):
        for u in range(2):
            acc = jnp.zeros((3136, 12), F32)
            for ey in range(3):
                wy, jy = _T2[t + ey]
                for ex in range(3):
                    wx, jx = _T2[u + ex]
                    xs = p6_ref[wy, wx, jy:jy + 56, jx:jx + 56, :].reshape(3136, 128)
                    acc = acc + jnp.dot(xs, w7_ref[ey * 3 + ex],
                                        preferred_element_type=F32)
            acc = acc + b7_ref[:]
            o_ref[0, t, u] = acc.reshape(56, 56, 12).astype(BF16)


def kernel(x, enc_w1, enc_b1, enc_w2, enc_b2, enc_w3, enc_b3, codebook,
           dec_w1, dec_b1, dec_wt1, dec_bt1, dec_wt2, dec_bt2):
    N = x.shape[0]

    w1 = _prep_conv1(enc_w1)
    w2 = _prep_conv2(enc_w2)
    w3 = _prep_s1(enc_w3)
    w5 = _prep_s1(dec_w1).astype(BF16)
    w6 = _prep_tconv(dec_wt1).astype(BF16)
    w7 = _prep_tconv_big(dec_wt2).astype(BF16)
    b1 = enc_b1.reshape(1, -1)
    b2 = enc_b2.reshape(1, -1)
    b3 = enc_b3.reshape(1, -1)
    b5 = dec_b1.reshape(1, -1)
    b6 = dec_bt1.reshape(1, -1)
    b7 = jnp.tile(dec_bt2, 4).reshape(1, -1)

    xh = jnp.transpose(x, (0, 2, 3, 1))
    xp = jnp.pad(xh, ((0, 0), (1, 3), (1, 3), (0, 0)))
    xs4 = xp.reshape(N, 57, 4, 57, 4, 3).transpose(0, 1, 3, 2, 4, 5)
    xs4 = jnp.pad(xs4.reshape(N, 57, 57, 48), ((0, 0), (0, 0), (0, 7), (0, 0)))

    idx3, loss_sum = pl.pallas_call(
        _enc_body,
        grid=(N,),
        in_specs=[
            pl.BlockSpec((1, 57, 64, 48), lambda n: (n, 0, 0, 0)),
            pl.BlockSpec((16, 48, 128), lambda n: (0, 0, 0)),
            pl.BlockSpec((1, 128), lambda n: (0, 0)),
            pl.BlockSpec((16, 128, 128), lambda n: (0, 0, 0)),
            pl.BlockSpec((1, 128), lambda n: (0, 0)),
            pl.BlockSpec((9, 128, 32), lambda n: (0, 0, 0)),
            pl.BlockSpec((1, 32), lambda n: (0, 0)),
            pl.BlockSpec((32, 1024), lambda n: (0, 0)),
        ],
        out_specs=[
            pl.BlockSpec((1, 3136, 1), lambda n: (n, 0, 0)),
            pl.BlockSpec((1, 1), lambda n: (0, 0)),
        ],
        out_shape=[
            jax.ShapeDtypeStruct((N, 3136, 1), jnp.int32),
            jax.ShapeDtypeStruct((1, 1), F32),
        ],
        scratch_shapes=[
            pltpu.VMEM((2, 2, 58, 64, 128), F32),
            pltpu.VMEM((58, 64, 128), F32),
        ],
    )(xs4, w1, b1, w2, b2, w3, b3, jnp.transpose(codebook))

    cb_pad = jnp.pad(codebook, ((0, 0), (0, 96)))
    q_flat = _sc_codebook_gather(cb_pad, idx3.reshape(N * 3136))
    q = q_flat.reshape(N, 56, 56, 128)

    out = pl.pallas_call(
        _dec_body,
        grid=(N,),
        in_specs=[
            pl.BlockSpec((1, 56, 56, 128), lambda n: (n, 0, 0, 0)),
            pl.BlockSpec((9, 32, 128), lambda n: (0, 0, 0)),
            pl.BlockSpec((1, 128), lambda n: (0, 0)),
            pl.BlockSpec((16, 128, 128), lambda n: (0, 0, 0)),
            pl.BlockSpec((1, 128), lambda n: (0, 0)),
            pl.BlockSpec((9, 128, 12), lambda n: (0, 0, 0)),
            pl.BlockSpec((1, 12), lambda n: (0, 0)),
        ],
        out_specs=pl.BlockSpec((1, 2, 2, 56, 56, 12), lambda n: (n, 0, 0, 0, 0, 0)),
        out_shape=jax.ShapeDtypeStruct((N, 2, 2, 56, 56, 12), BF16),
        scratch_shapes=[
            pltpu.VMEM((58, 64, 32), F32),
            pltpu.VMEM((58, 64, 128), BF16),
            pltpu.VMEM((2, 2, 58, 64, 128), BF16),
        ],
    )(q, w5, b5, w6, b6, w7, b7)

    o = out.astype(F32).reshape(N, 2, 2, 56, 56, 2, 2, 3)
    o = o.transpose(0, 3, 1, 5, 4, 2, 6, 7).reshape(N, 224, 224, 3)
    o = o.transpose(0, 3, 1, 2)

    loss = (1.25 / (N * 56 * 56 * 32)) * loss_sum[0, 0]
    return (o, loss)


# speedup vs baseline: 1.6777x; 1.0925x over previous
import functools

import jax
import jax.numpy as jnp
from jax import lax
from jax.experimental import pallas as pl
from jax.experimental.pallas import tpu as pltpu
from jax.experimental.pallas import tpu_sc as plsc

F32 = jnp.float32
BF16 = jnp.bfloat16


def _prep_conv1(w):
    wt = jnp.transpose(w, (2, 3, 1, 0))
    mats = []
    for v in range(2):
        for u in range(2):
            for dy in range(2):
                for dx in range(2):
                    m = jnp.zeros((4, 4, 3, 128), F32)
                    for p4y in range(4):
                        ky = 4 * dy + p4y - 2 * v
                        if not 0 <= ky < 4:
                            continue
                        for p4x in range(4):
                            kx = 4 * dx + p4x - 2 * u
                            if 0 <= kx < 4:
                                m = m.at[p4y, p4x].set(wt[ky, kx])
                    mats.append(m.reshape(48, 128))
    return jnp.stack(mats)


def _prep_conv2(w):
    wt = jnp.transpose(w, (2, 3, 1, 0))
    mats = []
    for py in range(2):
        for px in range(2):
            for dy in range(2):
                for dx in range(2):
                    mats.append(wt[2 * dy + py, 2 * dx + px])
    return jnp.stack(mats)


def _prep_s1(w):
    return jnp.transpose(w, (2, 3, 1, 0)).reshape(9, w.shape[1], w.shape[0])


def _prep_tconv(wt):
    mats = []
    for py in range(2):
        for px in range(2):
            for t0 in range(2):
                for t1 in range(2):
                    mats.append(wt[:, :, 3 - py - 2 * t0, 3 - px - 2 * t1])
    return jnp.stack(mats)


def _prep_tconv_big(wt):
    Cin, Cout = wt.shape[0], wt.shape[1]
    taps = []
    for ey in range(3):
        for ex in range(3):
            cols = []
            for qy in range(2):
                for qx in range(2):
                    t0, t1 = ey - qy, ex - qx
                    if t0 in (0, 1) and t1 in (0, 1):
                        cols.append(wt[:, :, 3 - qy - 2 * t0, 3 - qx - 2 * t1])
                    else:
                        cols.append(jnp.zeros((Cin, Cout), F32))
            taps.append(jnp.concatenate(cols, axis=1))
    return jnp.stack(taps)


def _enc_body(xs_ref, w1_ref, b1_ref, w2_ref, b2_ref, w3_ref, b3_ref,
              cbt_ref, idx_ref, loss_ref, p1_ref, h2_ref):
    n = pl.program_id(0)

    @pl.when(n == 0)
    def _():
        p1_ref[...] = jnp.zeros((2, 2, 58, 64, 128), F32)
        h2_ref[...] = jnp.zeros((58, 64, 128), F32)
        loss_ref[...] = jnp.zeros((1, 1), F32)

    accs1 = [[jnp.zeros((3136, 128), F32) for _ in range(2)] for _ in range(2)]
    for dy in range(2):
        for dx in range(2):
            xs = xs_ref[0, dy:dy + 56, dx:dx + 56, :].reshape(3136, 48)
            for v in range(2):
                for u in range(2):
                    accs1[v][u] = accs1[v][u] + jnp.dot(
                        xs, w1_ref[((v * 2 + u) * 2 + dy) * 2 + dx],
                        preferred_element_type=F32)
    for v in range(2):
        for u in range(2):
            acc = jnp.maximum(accs1[v][u] + b1_ref[:], 0.0)
            p1_ref[v, u, 1:57, 1:57, :] = acc.reshape(56, 56, 128)

    acc = jnp.zeros((3136, 128), F32)
    for py in range(2):
        for px in range(2):
            for dy in range(2):
                for dx in range(2):
                    xs = p1_ref[py ^ 1, px ^ 1, dy + py:dy + py + 56,
                                dx + px:dx + px + 56, :].reshape(3136, 128)
                    acc = acc + jnp.dot(xs, w2_ref[((py * 2 + px) * 2 + dy) * 2 + dx],
                                        preferred_element_type=F32)
    acc = jnp.maximum(acc + b2_ref[:], 0.0)
    h2_ref[1:57, 1:57, :] = acc.reshape(56, 56, 128)

    acc = jnp.zeros((3136, 32), F32)
    for ky in range(3):
        for kx in range(3):
            xs = h2_ref[ky:ky + 56, kx:kx + 56, :].reshape(3136, 128)
            acc = acc + jnp.dot(xs, w3_ref[ky * 3 + kx], preferred_element_type=F32)
    z = acc + b3_ref[:]

    cb_sq = jnp.sum(cbt_ref[:] ** 2, axis=0, keepdims=True)
    loss_part = jnp.sum(z * z).reshape(1, 1)
    for c in range(4):
        zc = z[c * 784:(c + 1) * 784, :]
        scores = cb_sq - 2.0 * jnp.dot(zc, cbt_ref[:], preferred_element_type=F32)
        iota = jax.lax.broadcasted_iota(jnp.int32, (784, 1024), 1)
        m = jnp.min(scores, axis=1, keepdims=True)
        idx = jnp.min(jnp.where(scores == m, iota, 1024), axis=1, keepdims=True)
        loss_part = loss_part + jnp.sum(m).reshape(1, 1)
        idx_ref[0, c * 784:(c + 1) * 784, :] = idx
    loss_ref[:] = loss_ref[:] + loss_part


def _sc_codebook_gather(codebook, idx_flat):
    B = idx_flat.shape[0]
    D = codebook.shape[1]
    info = plsc.get_sparse_core_info()
    NW = info.num_cores * info.num_subcores
    n_chunks = 2
    assert B % (8 * NW * n_chunks) == 0
    b_per_c = B // (NW * n_chunks)
    mesh = plsc.VectorSubcoreMesh(core_axis_name="c", subcore_axis_name="s")

    @functools.partial(
        pl.kernel, mesh=mesh,
        out_type=jax.ShapeDtypeStruct((B, D), F32),
        scratch_types=[
            pltpu.VMEM((b_per_c,), jnp.int32),
            pltpu.VMEM((b_per_c, D), F32),
            pltpu.SemaphoreType.DMA,
        ],
    )
    def k(table_hbm, idx_hbm, out_hbm, idx_v, rows_v, sem):
        wid = lax.axis_index("s") * info.num_cores + lax.axis_index("c")
        for c in range(n_chunks):
            base = (wid * n_chunks + c) * b_per_c
            pltpu.sync_copy(idx_hbm.at[pl.ds(base, b_per_c)], idx_v)
            pltpu.async_copy(table_hbm.at[idx_v], rows_v, sem).wait()
            pltpu.sync_copy(rows_v, out_hbm.at[pl.ds(base, b_per_c)])

    return k(codebook, idx_flat)


_T2 = {0: (1, 0), 1: (0, 1), 2: (1, 1), 3: (0, 2)}


def _dec_body(q_ref, w5_ref, b5_ref, w6_ref, b6_ref, w7_ref, b7_ref,
              o_ref, qp_ref, h5_ref, p6_ref):
    n = pl.program_id(0)

    @pl.when(n == 0)
    def _():
        qp_ref[...] = jnp.zeros((58, 64, 32), F32)
        h5_ref[...] = jnp.zeros((58, 64, 128), BF16)
        p6_ref[...] = jnp.zeros((2, 2, 58, 64, 128), BF16)

    qp_ref[1:57, 1:57, :] = q_ref[0, :, :, 0:32]

    acc = jnp.zeros((3136, 128), F32)
    for ky in range(3):
        for kx in range(3):
            xs = qp_ref[ky:ky + 56, kx:kx + 56, :].reshape(3136, 32)
            acc = acc + jnp.dot(xs.astype(BF16), w5_ref[ky * 3 + kx],
                                preferred_element_type=F32)
    acc = jnp.maximum(acc + b5_ref[:], 0.0)
    h5_ref[1:57, 1:57, :] = acc.reshape(56, 56, 128).astype(BF16)

    accs6 = [[jnp.zeros((3136, 128), F32) for _ in range(2)] for _ in range(2)]
    for s in range(3):
        for sx in range(3):
            xs = h5_ref[s:s + 56, sx:sx + 56, :].reshape(3136, 128)
            for py in (p for p in range(2) if 0 <= s - p <= 1):
                for px in (p for p in range(2) if 0 <= sx - p <= 1):
                    accs6[py][px] = accs6[py][px] + jnp.dot(
                        xs, w6_ref[((py * 2 + px) * 2 + (s - py)) * 2 + (sx - px)],
                        preferred_element_type=F32)
    for py in range(2):
        for px in range(2):
            acc = jnp.maximum(accs6[py][px] + b6_ref[:], 0.0)
            p6_ref[py, px, 1:57, 1:57, :] = acc.reshape(56, 56, 128).astype(BF16)

    accs7 = [[jnp.zeros((3136, 12), F32) for _ in range(2)] for _ in range(2)]
    for a in range(4):
        wy, jy = _T2[a]
        for b in range(4):
            wx, jx = _T2[b]
            xs = p6_ref[wy, wx, jy:jy + 56, jx:jx + 56, :].reshape(3136, 128)
            for t in (t_ for t_ in range(2) if 0 <= a - t_ <= 2):
                for u in (u_ for u_ in range(2) if 0 <= b - u_ <= 2):
                    accs7[t][u] = accs7[t][u] + jnp.dot(
                        xs, w7_ref[(a - t) * 3 + (b - u)],
                        preferred_element_type=F32)
    for t in range(2):
        for u in range(2):
            acc = accs7[t][u] + b7_ref[:]
            o_ref[0, t, u] = acc.reshape(56, 56, 12).astype(BF16)


def kernel(x, enc_w1, enc_b1, enc_w2, enc_b2, enc_w3, enc_b3, codebook,
           dec_w1, dec_b1, dec_wt1, dec_bt1, dec_wt2, dec_bt2):
    N = x.shape[0]

    w1 = _prep_conv1(enc_w1)
    w2 = _prep_conv2(enc_w2)
    w3 = _prep_s1(enc_w3)
    w5 = _prep_s1(dec_w1).astype(BF16)
    w6 = _prep_tconv(dec_wt1).astype(BF16)
    w7 = _prep_tconv_big(dec_wt2).astype(BF16)
    b1 = enc_b1.reshape(1, -1)
    b2 = enc_b2.reshape(1, -1)
    b3 = enc_b3.reshape(1, -1)
    b5 = dec_b1.reshape(1, -1)
    b6 = dec_bt1.reshape(1, -1)
    b7 = jnp.tile(dec_bt2, 4).reshape(1, -1)

    xh = jnp.transpose(x, (0, 2, 3, 1))
    xp = jnp.pad(xh, ((0, 0), (1, 3), (1, 3), (0, 0)))
    xs4 = xp.reshape(N, 57, 4, 57, 4, 3).transpose(0, 1, 3, 2, 4, 5)
    xs4 = jnp.pad(xs4.reshape(N, 57, 57, 48), ((0, 0), (0, 0), (0, 7), (0, 0)))

    idx3, loss_sum = pl.pallas_call(
        _enc_body,
        grid=(N,),
        in_specs=[
            pl.BlockSpec((1, 57, 64, 48), lambda n: (n, 0, 0, 0)),
            pl.BlockSpec((16, 48, 128), lambda n: (0, 0, 0)),
            pl.BlockSpec((1, 128), lambda n: (0, 0)),
            pl.BlockSpec((16, 128, 128), lambda n: (0, 0, 0)),
            pl.BlockSpec((1, 128), lambda n: (0, 0)),
            pl.BlockSpec((9, 128, 32), lambda n: (0, 0, 0)),
            pl.BlockSpec((1, 32), lambda n: (0, 0)),
            pl.BlockSpec((32, 1024), lambda n: (0, 0)),
        ],
        out_specs=[
            pl.BlockSpec((1, 3136, 1), lambda n: (n, 0, 0)),
            pl.BlockSpec((1, 1), lambda n: (0, 0)),
        ],
        out_shape=[
            jax.ShapeDtypeStruct((N, 3136, 1), jnp.int32),
            jax.ShapeDtypeStruct((1, 1), F32),
        ],
        scratch_shapes=[
            pltpu.VMEM((2, 2, 58, 64, 128), F32),
            pltpu.VMEM((58, 64, 128), F32),
        ],
    )(xs4, w1, b1, w2, b2, w3, b3, jnp.transpose(codebook))

    cb_pad = jnp.pad(codebook, ((0, 0), (0, 96)))
    q_flat = _sc_codebook_gather(cb_pad, idx3.reshape(N * 3136))
    q = q_flat.reshape(N, 56, 56, 128)

    out = pl.pallas_call(
        _dec_body,
        grid=(N,),
        in_specs=[
            pl.BlockSpec((1, 56, 56, 128), lambda n: (n, 0, 0, 0)),
            pl.BlockSpec((9, 32, 128), lambda n: (0, 0, 0)),
            pl.BlockSpec((1, 128), lambda n: (0, 0)),
            pl.BlockSpec((16, 128, 128), lambda n: (0, 0, 0)),
            pl.BlockSpec((1, 128), lambda n: (0, 0)),
            pl.BlockSpec((9, 128, 12), lambda n: (0, 0, 0)),
            pl.BlockSpec((1, 12), lambda n: (0, 0)),
        ],
        out_specs=pl.BlockSpec((1, 2, 2, 56, 56, 12), lambda n: (n, 0, 0, 0, 0, 0)),
        out_shape=jax.ShapeDtypeStruct((N, 2, 2, 56, 56, 12), BF16),
        scratch_shapes=[
            pltpu.VMEM((58, 64, 32), F32),
            pltpu.VMEM((58, 64, 128), BF16),
            pltpu.VMEM((2, 2, 58, 64, 128), BF16),
        ],
    )(q, w5, b5, w6, b6, w7, b7)

    o = out.astype(F32).reshape(N, 2, 2, 56, 56, 2, 2, 3)
    o = o.transpose(0, 3, 1, 5, 4, 2, 6, 7).reshape(N, 224, 224, 3)
    o = o.transpose(0, 3, 1, 2)

    loss = (1.25 / (N * 56 * 56 * 32)) * loss_sum[0, 0]
    return (o, loss)


# speedup vs baseline: 1.8153x; 1.0820x over previous
import functools

import jax
import jax.numpy as jnp
from jax import lax
from jax.experimental import pallas as pl
from jax.experimental.pallas import tpu as pltpu
from jax.experimental.pallas import tpu_sc as plsc

F32 = jnp.float32
BF16 = jnp.bfloat16


def _prep_conv1(w):
    wt = jnp.transpose(w, (2, 3, 1, 0))
    taps = []
    for dy in range(2):
        for dx in range(2):
            cols = []
            for v in range(2):
                for u in range(2):
                    m = jnp.zeros((4, 4, 3, 128), F32)
                    for p4y in range(4):
                        ky = 4 * dy + p4y - 2 * v
                        if not 0 <= ky < 4:
                            continue
                        for p4x in range(4):
                            kx = 4 * dx + p4x - 2 * u
                            if 0 <= kx < 4:
                                m = m.at[p4y, p4x].set(wt[ky, kx])
                    cols.append(m.reshape(48, 128))
            taps.append(jnp.concatenate(cols, axis=1))
    return jnp.stack(taps)


def _prep_conv2(w):
    wt = jnp.transpose(w, (2, 3, 1, 0))
    mats = []
    for py in range(2):
        for px in range(2):
            for dy in range(2):
                for dx in range(2):
                    mats.append(wt[2 * dy + py, 2 * dx + px])
    return jnp.stack(mats)


def _prep_s1(w):
    return jnp.transpose(w, (2, 3, 1, 0)).reshape(9, w.shape[1], w.shape[0])


_TCONV_VALID = {}
for _s in range(3):
    for _sx in range(3):
        _TCONV_VALID[(_s, _sx)] = [
            (py, px) for py in range(2) if 0 <= _s - py <= 1
            for px in range(2) if 0 <= _sx - px <= 1]


def _prep_tconv(wt):
    mats = {}
    for (s, sx), valid in _TCONV_VALID.items():
        cols = [wt[:, :, 3 - py - 2 * (s - py), 3 - px - 2 * (sx - px)]
                for py, px in valid]
        mats[(s, sx)] = jnp.concatenate(cols, axis=1)
    return mats


def _prep_tconv2_packed(wt):
    Cin, Cout = wt.shape[0], wt.shape[1]
    zero = jnp.zeros((Cin, Cout), F32)
    mats = []
    for a in range(4):
        for b in range(4):
            cols = []
            for t in range(2):
                for u in range(2):
                    for qy in range(2):
                        for qx in range(2):
                            ey, ex = a - t, b - u
                            ok = (0 <= ey <= 2 and 0 <= ex <= 2
                                  and ey - qy in (0, 1) and ex - qx in (0, 1))
                            cols.append(
                                wt[:, :, 3 + qy - 2 * ey, 3 + qx - 2 * ex]
                                if ok else zero)
            mats.append(jnp.concatenate(cols, axis=1))
    return jnp.stack(mats)


def _enc_body(xs_ref, w1_ref, b1_ref, w2_ref, b2_ref, w3_ref, b3_ref,
              cbt_ref, idx_ref, loss_ref, p1_ref, h2_ref):
    n = pl.program_id(0)

    @pl.when(n == 0)
    def _():
        p1_ref[...] = jnp.zeros((2, 2, 58, 64, 128), F32)
        h2_ref[...] = jnp.zeros((58, 64, 128), F32)
        loss_ref[...] = jnp.zeros((1, 1), F32)

    acc1 = jnp.zeros((3136, 512), F32)
    for dy in range(2):
        for dx in range(2):
            xs = xs_ref[0, dy:dy + 56, dx:dx + 56, :].reshape(3136, 48)
            acc1 = acc1 + jnp.dot(xs, w1_ref[dy * 2 + dx],
                                  preferred_element_type=F32)
    acc1 = jnp.maximum(acc1 + b1_ref[:], 0.0)
    for v in range(2):
        for u in range(2):
            p = (v * 2 + u) * 128
            p1_ref[v, u, 1:57, 1:57, :] = acc1[:, p:p + 128].reshape(56, 56, 128)

    acc = jnp.zeros((3136, 128), F32)
    for py in range(2):
        for px in range(2):
            for dy in range(2):
                for dx in range(2):
                    xs = p1_ref[py ^ 1, px ^ 1, dy + py:dy + py + 56,
                                dx + px:dx + px + 56, :].reshape(3136, 128)
                    acc = acc + jnp.dot(xs, w2_ref[((py * 2 + px) * 2 + dy) * 2 + dx],
                                        preferred_element_type=F32)
    acc = jnp.maximum(acc + b2_ref[:], 0.0)
    h2_ref[1:57, 1:57, :] = acc.reshape(56, 56, 128)

    acc = jnp.zeros((3136, 32), F32)
    for ky in range(3):
        for kx in range(3):
            xs = h2_ref[ky:ky + 56, kx:kx + 56, :].reshape(3136, 128)
            acc = acc + jnp.dot(xs, w3_ref[ky * 3 + kx], preferred_element_type=F32)
    z = acc + b3_ref[:]

    cb_sq = jnp.sum(cbt_ref[:] ** 2, axis=0, keepdims=True)
    loss_part = jnp.sum(z * z).reshape(1, 1)
    for c in range(4):
        zc = z[c * 784:(c + 1) * 784, :]
        scores = cb_sq - 2.0 * jnp.dot(zc, cbt_ref[:], preferred_element_type=F32)
        iota = jax.lax.broadcasted_iota(jnp.int32, (784, 1024), 1)
        m = jnp.min(scores, axis=1, keepdims=True)
        idx = jnp.min(jnp.where(scores == m, iota, 1024), axis=1, keepdims=True)
        loss_part = loss_part + jnp.sum(m).reshape(1, 1)
        idx_ref[0, c * 784:(c + 1) * 784, :] = idx
    loss_ref[:] = loss_ref[:] + loss_part


def _sc_codebook_gather(codebook, idx_flat):
    B = idx_flat.shape[0]
    D = codebook.shape[1]
    info = plsc.get_sparse_core_info()
    NW = info.num_cores * info.num_subcores
    n_chunks = 2
    assert B % (8 * NW * n_chunks) == 0
    b_per_c = B // (NW * n_chunks)
    mesh = plsc.VectorSubcoreMesh(core_axis_name="c", subcore_axis_name="s")

    @functools.partial(
        pl.kernel, mesh=mesh,
        out_type=jax.ShapeDtypeStruct((B, D), F32),
        scratch_types=[
            pltpu.VMEM((b_per_c,), jnp.int32),
            pltpu.VMEM((b_per_c, D), F32),
            pltpu.SemaphoreType.DMA,
        ],
    )
    def k(table_hbm, idx_hbm, out_hbm, idx_v, rows_v, sem):
        wid = lax.axis_index("s") * info.num_cores + lax.axis_index("c")
        for c in range(n_chunks):
            base = (wid * n_chunks + c) * b_per_c
            pltpu.sync_copy(idx_hbm.at[pl.ds(base, b_per_c)], idx_v)
            pltpu.async_copy(table_hbm.at[idx_v], rows_v, sem).wait()
            pltpu.sync_copy(rows_v, out_hbm.at[pl.ds(base, b_per_c)])

    return k(codebook, idx_flat)


_T2 = {0: (1, 0), 1: (0, 1), 2: (1, 1), 3: (0, 2)}
_CORNERS = [(0, 0), (0, 2), (2, 0), (2, 2)]
_EDGES = [(0, 1), (1, 0), (1, 2), (2, 1)]


def _dec_body(q_ref, w5_ref, b5_ref, w6k_ref, w6e_ref, w6c_ref, b6_ref,
              w7_ref, b7_ref, o_ref, qp_ref, h5_ref, p6_ref):
    n = pl.program_id(0)

    @pl.when(n == 0)
    def _():
        qp_ref[...] = jnp.zeros((58, 64, 32), F32)
        h5_ref[...] = jnp.zeros((58, 64, 128), BF16)
        p6_ref[...] = jnp.zeros((2, 2, 58, 64, 128), BF16)

    qp_ref[1:57, 1:57, :] = q_ref[0, :, :, 0:32]

    acc = jnp.zeros((3136, 128), F32)
    for ky in range(3):
        for kx in range(3):
            xs = qp_ref[ky:ky + 56, kx:kx + 56, :].reshape(3136, 32)
            acc = acc + jnp.dot(xs.astype(BF16), w5_ref[ky * 3 + kx],
                                preferred_element_type=F32)
    acc = jnp.maximum(acc + b5_ref[:], 0.0)
    h5_ref[1:57, 1:57, :] = acc.reshape(56, 56, 128).astype(BF16)

    accs6 = [[jnp.zeros((3136, 128), F32) for _ in range(2)] for _ in range(2)]
    for s in range(3):
        for sx in range(3):
            valid = _TCONV_VALID[(s, sx)]
            if (s, sx) == (1, 1):
                w = w6c_ref[:]
            elif len(valid) == 2:
                w = w6e_ref[_EDGES.index((s, sx))]
            else:
                w = w6k_ref[_CORNERS.index((s, sx))]
            xs = h5_ref[s:s + 56, sx:sx + 56, :].reshape(3136, 128)
            r = jnp.dot(xs, w, preferred_element_type=F32)
            for i, (py, px) in enumerate(valid):
                accs6[py][px] = accs6[py][px] + r[:, i * 128:(i + 1) * 128]
    for py in range(2):
        for px in range(2):
            acc = jnp.maximum(accs6[py][px] + b6_ref[:], 0.0)
            p6_ref[py, px, 1:57, 1:57, :] = acc.reshape(56, 56, 128).astype(BF16)

    acc48 = jnp.zeros((3136, 48), F32)
    for a in range(4):
        wy, jy = _T2[a]
        for b in range(4):
            wx, jx = _T2[b]
            xs = p6_ref[wy, wx, jy:jy + 56, jx:jx + 56, :].reshape(3136, 128)
            acc48 = acc48 + jnp.dot(xs, w7_ref[a * 4 + b],
                                    preferred_element_type=F32)
    acc48 = acc48 + b7_ref[:]
    o_ref[0] = acc48.reshape(56, 56, 48).astype(BF16)


def kernel(x, enc_w1, enc_b1, enc_w2, enc_b2, enc_w3, enc_b3, codebook,
           dec_w1, dec_b1, dec_wt1, dec_bt1, dec_wt2, dec_bt2):
    N = x.shape[0]

    w1 = _prep_conv1(enc_w1)
    w2 = _prep_conv2(enc_w2)
    w3 = _prep_s1(enc_w3)
    w5 = _prep_s1(dec_w1).astype(BF16)
    w6m = _prep_tconv(dec_wt1)
    w6k = jnp.stack([w6m[c] for c in _CORNERS]).astype(BF16)
    w6e = jnp.stack([w6m[e] for e in _EDGES]).astype(BF16)
    w6c = w6m[(1, 1)].astype(BF16)
    w7 = _prep_tconv2_packed(dec_wt2).astype(BF16)
    b1 = jnp.tile(enc_b1, 4).reshape(1, -1)
    b2 = enc_b2.reshape(1, -1)
    b3 = enc_b3.reshape(1, -1)
    b5 = dec_b1.reshape(1, -1)
    b6 = dec_bt1.reshape(1, -1)
    b7 = jnp.tile(dec_bt2, 16).reshape(1, -1)

    xh = jnp.transpose(x, (0, 2, 3, 1))
    xp = jnp.pad(xh, ((0, 0), (1, 3), (1, 3), (0, 0)))
    xs4 = xp.reshape(N, 57, 4, 57, 4, 3).transpose(0, 1, 3, 2, 4, 5)
    xs4 = jnp.pad(xs4.reshape(N, 57, 57, 48), ((0, 0), (0, 0), (0, 7), (0, 0)))

    idx3, loss_sum = pl.pallas_call(
        _enc_body,
        grid=(N,),
        in_specs=[
            pl.BlockSpec((1, 57, 64, 48), lambda n: (n, 0, 0, 0)),
            pl.BlockSpec((4, 48, 512), lambda n: (0, 0, 0)),
            pl.BlockSpec((1, 512), lambda n: (0, 0)),
            pl.BlockSpec((16, 128, 128), lambda n: (0, 0, 0)),
            pl.BlockSpec((1, 128), lambda n: (0, 0)),
            pl.BlockSpec((9, 128, 32), lambda n: (0, 0, 0)),
            pl.BlockSpec((1, 32), lambda n: (0, 0)),
            pl.BlockSpec((32, 1024), lambda n: (0, 0)),
        ],
        out_specs=[
            pl.BlockSpec((1, 3136, 1), lambda n: (n, 0, 0)),
            pl.BlockSpec((1, 1), lambda n: (0, 0)),
        ],
        out_shape=[
            jax.ShapeDtypeStruct((N, 3136, 1), jnp.int32),
            jax.ShapeDtypeStruct((1, 1), F32),
        ],
        scratch_shapes=[
            pltpu.VMEM((2, 2, 58, 64, 128), F32),
            pltpu.VMEM((58, 64, 128), F32),
        ],
    )(xs4, w1, b1, w2, b2, w3, b3, jnp.transpose(codebook))

    cb_pad = jnp.pad(codebook, ((0, 0), (0, 96)))
    q_flat = _sc_codebook_gather(cb_pad, idx3.reshape(N * 3136))
    q = q_flat.reshape(N, 56, 56, 128)

    out = pl.pallas_call(
        _dec_body,
        grid=(N,),
        in_specs=[
            pl.BlockSpec((1, 56, 56, 128), lambda n: (n, 0, 0, 0)),
            pl.BlockSpec((9, 32, 128), lambda n: (0, 0, 0)),
            pl.BlockSpec((1, 128), lambda n: (0, 0)),
            pl.BlockSpec((4, 128, 128), lambda n: (0, 0, 0)),
            pl.BlockSpec((4, 128, 256), lambda n: (0, 0, 0)),
            pl.BlockSpec((128, 512), lambda n: (0, 0)),
            pl.BlockSpec((1, 128), lambda n: (0, 0)),
            pl.BlockSpec((16, 128, 48), lambda n: (0, 0, 0)),
            pl.BlockSpec((1, 48), lambda n: (0, 0)),
        ],
        out_specs=pl.BlockSpec((1, 56, 56, 48), lambda n: (n, 0, 0, 0)),
        out_shape=jax.ShapeDtypeStruct((N, 56, 56, 48), BF16),
        scratch_shapes=[
            pltpu.VMEM((58, 64, 32), F32),
            pltpu.VMEM((58, 64, 128), BF16),
            pltpu.VMEM((2, 2, 58, 64, 128), BF16),
        ],
    )(q, w5, b5, w6k, w6e, w6c, b6, w7, b7)

    o = out.astype(F32).reshape(N, 56, 56, 2, 2, 2, 2, 3)
    o = o.transpose(0, 1, 3, 5, 2, 4, 6, 7).reshape(N, 224, 224, 3)
    o = o.transpose(0, 3, 1, 2)

    loss = (1.25 / (N * 56 * 56 * 32)) * loss_sum[0, 0]
    return (o, loss)
